# Initial kernel scaffold; baseline (speedup 1.0000x reference)
import jax
import jax.numpy as jnp
from jax.experimental import pallas as pl


def kernel(tokens, wte_weight, learned_embedding):
    raise NotImplementedError("write your pallas kernel here")


# speedup vs baseline: 7.8814x; 7.8814x over previous
import functools

import jax
import jax.numpy as jnp
from jax import lax
from jax.experimental import pallas as pl
from jax.experimental.pallas import tpu as pltpu
from jax.experimental.pallas import tpu_sc as plsc

VOCAB = 100000
D = 64
B = 16384
SEQ = 200
NT = 20
S1 = 10

_info = plsc.get_sparse_core_info()
_NC = _info.num_cores
_NS = _info.num_subcores
_NW = _NC * _NS
_ROWS_PER_W = B // _NW
_G = 4
_CW = _G * SEQ
_IDX_CHUNK = 128


@functools.partial(
    pl.kernel,
    mesh=plsc.VectorSubcoreMesh(core_axis_name="c", subcore_axis_name="s"),
    out_type=jax.ShapeDtypeStruct((B * SEQ, D), jnp.float32),
    compiler_params=pltpu.CompilerParams(use_tc_tiling_on_sc=False),
    scratch_types=[
        pltpu.VMEM((_CW,), jnp.int32),
        pltpu.VMEM((_CW, D), jnp.float32),
        pltpu.SemaphoreType.DMA,
    ],
)
def _prompt_embed(tok_hbm, table_hbm, out_hbm, idx_v, gbuf, sem):
    wid = lax.axis_index("s") * _NC + lax.axis_index("c")
    base_row = wid * _ROWS_PER_W

    iota = lax.iota(jnp.int32, 16)

    def chunk_body(c, carry):
        row0 = base_row + c * _G
        pltpu.sync_copy(tok_hbm.at[pl.ds(row0 * SEQ, _CW)], idx_v)
        for r in range(_G):
            rb = r * SEQ
            w0 = ((rb + 1) // 16) * 16
            g0 = idx_v[pl.ds(w0, 16)]
            g1 = idx_v[pl.ds(w0 + 16, 16)]
            t21 = g1[rb + NT + 1 - (w0 + 16)]
            for w, g in ((w0, g0), (w0 + 16, g1)):
                s = (w - rb) + iota
                in_r = (s >= 1) & (s <= NT + 1)
                cval = VOCAB + jnp.where(s <= S1, s - 1, s - 2)
                new = jnp.where(in_r, jnp.where(s == S1 + 1, t21, cval), g)
                idx_v[pl.ds(w, 16)] = new
        copies = []
        for k in range(0, _CW, _IDX_CHUNK):
            n = min(_IDX_CHUNK, _CW - k)
            copies.append(pltpu.async_copy(
                table_hbm.at[idx_v.at[pl.ds(k, n)]],
                gbuf.at[pl.ds(k, n)], sem))
        for cp in copies:
            cp.wait()
        pltpu.sync_copy(gbuf, out_hbm.at[pl.ds(row0 * SEQ, _CW)])
        return carry

    lax.fori_loop(0, _ROWS_PER_W // _G, chunk_body, 0)


def kernel(tokens, wte_weight, learned_embedding):
    table = jnp.concatenate([wte_weight, learned_embedding], axis=0)
    out = _prompt_embed(tokens.reshape(B * SEQ), table)
    return out.reshape(B, SEQ, D)
